# baseline (speedup 1.0000x reference)
import functools

import jax
import jax.numpy as jnp
from jax import lax
from jax.experimental import pallas as pl
from jax.experimental.pallas import tpu as pltpu
from jax.experimental.pallas import tpu_sc as plsc

_T = 1024
_P = 56
_D = 64
_B = 16384
_NW = 32
_BPW = _B // _NW
_NBLK = _P * (_BPW // 128)
_TPW = _T // _NW

_sc_mesh = plsc.VectorSubcoreMesh(core_axis_name="c", subcore_axis_name="s")
_sc_params = pltpu.CompilerParams(use_tc_tiling_on_sc=False,
                                  needs_layout_passes=False)


@functools.partial(
    pl.kernel,
    mesh=_sc_mesh,
    out_type=jax.ShapeDtypeStruct((_T * _P, _D), jnp.float32),
    scratch_types=[
        pltpu.VMEM((_TPW, _D), jnp.float32),
        pltpu.VMEM((_P, _D), jnp.float32),
        pltpu.VMEM((_TPW * _P, _D), jnp.float32),
    ],
    compiler_params=_sc_params,
)
def _sc_fuse(emb_hbm, pos_hbm, fused_hbm, emb_v, pos_v, f_v):
    wid = lax.axis_index("s") * 2 + lax.axis_index("c")
    pltpu.sync_copy(emb_hbm.at[pl.ds(wid * _TPW, _TPW)], emb_v)
    pltpu.sync_copy(pos_hbm, pos_v)

    def t_body(tl, carry):
        erow = emb_v.at[tl]
        e = [erow[pl.ds(i * 16, 16)] for i in range(_D // 16)]

        def p_body(p, c2):
            frow = f_v.at[tl * _P + p]
            prow = pos_v.at[p]
            for i in range(_D // 16):
                frow[pl.ds(i * 16, 16)] = e[i] + prow[pl.ds(i * 16, 16)]
            return c2

        lax.fori_loop(0, _P, p_body, 0)
        return carry

    lax.fori_loop(0, _TPW, t_body, 0)
    pltpu.sync_copy(f_v, fused_hbm.at[pl.ds(wid * _TPW * _P, _TPW * _P)])


@functools.partial(
    pl.kernel,
    mesh=_sc_mesh,
    out_type=jax.ShapeDtypeStruct((_P, _D // 8, _B // 128, 8, 128),
                                  jnp.float32),
    scratch_types=[
        pltpu.VMEM((_P, _BPW), jnp.int32),
        pltpu.VMEM((4, 128), jnp.int32),
        pltpu.VMEM((4, 128, _D), jnp.float32),
        pltpu.VMEM((4, _D // 8, 8, 128), jnp.float32),
        pltpu.SemaphoreType.DMA,
        pltpu.SemaphoreType.DMA,
        pltpu.SemaphoreType.DMA,
        pltpu.SemaphoreType.DMA,
        pltpu.SemaphoreType.DMA,
        pltpu.SemaphoreType.DMA,
        pltpu.SemaphoreType.DMA,
        pltpu.SemaphoreType.DMA,
    ],
    compiler_params=_sc_params,
)
def _sc_embed(tok_hbm, fused_hbm, out_hbm, tok_v, idx_v, rows_v, trans_v,
              gsem0, gsem1, gsem2, gsem3, osem0, osem1, osem2, osem3):
    wid = lax.axis_index("s") * 2 + lax.axis_index("c")
    gsem = (gsem0, gsem1, gsem2, gsem3)
    osem = (osem0, osem1, osem2, osem3)

    pltpu.sync_copy(tok_hbm.at[pl.ds(0, _P), pl.ds(wid * _BPW, _BPW)], tok_v)

    def build_idx_and_fire(p, bt):
        pv = jnp.full((16,), p, dtype=jnp.int32)
        trow = tok_v.at[p]
        for s in range(8):
            t16 = trow[pl.ds(bt * 128 + s * 16, 16)]
            idx_v[bt, pl.ds(s * 16, 16)] = t16 * _P + pv
        pltpu.async_copy(fused_hbm.at[idx_v.at[bt]], rows_v.at[bt], gsem[bt])

    def wait_g(bt):
        pltpu.make_async_copy(
            fused_hbm.at[pl.ds(0, 128)], rows_v.at[bt], gsem[bt]).wait()

    def transpose(bt):
        rows = rows_v.at[bt]
        rowbase = [
            lax.iota(jnp.int32, 16) + (s * 16) for s in range(8)
        ]
        for d in range(_D):
            col = jnp.full((16,), d, dtype=jnp.int32)
            for s in range(8):
                v = plsc.load_gather(rows, [rowbase[s], col])
                trans_v[bt, d // 8, d % 8, pl.ds(s * 16, 16)] = v

    def fire_out(p, bt):
        gbt = wid * 4 + bt
        for dblk in range(_D // 8):
            pltpu.async_copy(
                trans_v.at[bt].at[dblk],
                out_hbm.at[p].at[dblk].at[gbt],
                osem[bt],
            )

    def wait_o(bt):
        for dblk in range(_D // 8):
            pltpu.make_async_copy(
                trans_v.at[bt].at[dblk], out_hbm.at[0].at[dblk].at[0],
                osem[bt]).wait()

    for bt in range(4):
        build_idx_and_fire(0, bt)

    def body(p, carry):
        for bt in range(4):
            wait_g(bt)

            @pl.when(p >= 1)
            def _():
                wait_o(bt)

            transpose(bt)
            fire_out(p, bt)

            @pl.when(p <= _P - 2)
            def _():
                build_idx_and_fire(p + 1, bt)

        return carry

    lax.fori_loop(0, _P, body, 0)
    for bt in range(4):
        wait_o(bt)


def kernel(tokens, embedding, position_embedding):
    tok_t = tokens.astype(jnp.int32).T
    fused = _sc_fuse(embedding, position_embedding)
    out5 = _sc_embed(tok_t, fused)
    return out5.transpose(2, 4, 0, 1, 3).reshape(_B, _P, _D)


# speedup vs baseline: 2.4512x; 2.4512x over previous
import functools

import jax
import jax.numpy as jnp
from jax import lax
from jax.experimental import pallas as pl
from jax.experimental.pallas import tpu as pltpu
from jax.experimental.pallas import tpu_sc as plsc

_T = 1024
_P = 56
_D = 64
_B = 16384
_NW = 32
_BPW = _B // _NW
_NBLK = _P * (_BPW // 128)
_TPW = _T // _NW

_sc_mesh = plsc.VectorSubcoreMesh(core_axis_name="c", subcore_axis_name="s")
_sc_params = pltpu.CompilerParams(use_tc_tiling_on_sc=False,
                                  needs_layout_passes=False)


@functools.partial(
    pl.kernel,
    mesh=_sc_mesh,
    out_type=jax.ShapeDtypeStruct((_T * _P, _D), jnp.float32),
    scratch_types=[
        pltpu.VMEM((_TPW, _D), jnp.float32),
        pltpu.VMEM((_P, _D), jnp.float32),
        pltpu.VMEM((_TPW * _P, _D), jnp.float32),
    ],
    compiler_params=_sc_params,
)
def _sc_fuse(emb_hbm, pos_hbm, fused_hbm, emb_v, pos_v, f_v):
    wid = lax.axis_index("s") * 2 + lax.axis_index("c")
    pltpu.sync_copy(emb_hbm.at[pl.ds(wid * _TPW, _TPW)], emb_v)
    pltpu.sync_copy(pos_hbm, pos_v)

    def t_body(tl, carry):
        erow = emb_v.at[tl]
        e = [erow[pl.ds(i * 16, 16)] for i in range(_D // 16)]

        def p_body(p, c2):
            frow = f_v.at[tl * _P + p]
            prow = pos_v.at[p]
            for i in range(_D // 16):
                frow[pl.ds(i * 16, 16)] = e[i] + prow[pl.ds(i * 16, 16)]
            return c2

        lax.fori_loop(0, _P, p_body, 0)
        return carry

    lax.fori_loop(0, _TPW, t_body, 0)
    pltpu.sync_copy(f_v, fused_hbm.at[pl.ds(wid * _TPW * _P, _TPW * _P)])


@functools.partial(
    pl.kernel,
    mesh=_sc_mesh,
    out_type=jax.ShapeDtypeStruct((_P, _D // 8, _B // 128, 1024),
                                  jnp.float32),
    scratch_types=[
        pltpu.VMEM((_P, _BPW), jnp.int32),
        pltpu.VMEM((4, 128), jnp.int32),
        pltpu.VMEM((4, 128, _D), jnp.float32),
        pltpu.VMEM((4, _D * 128), jnp.float32),
        pltpu.SemaphoreType.DMA,
        pltpu.SemaphoreType.DMA,
        pltpu.SemaphoreType.DMA,
        pltpu.SemaphoreType.DMA,
        pltpu.SemaphoreType.DMA,
        pltpu.SemaphoreType.DMA,
        pltpu.SemaphoreType.DMA,
        pltpu.SemaphoreType.DMA,
    ],
    compiler_params=_sc_params,
)
def _sc_embed(tok_hbm, fused_hbm, out_hbm, tok_v, idx_v, rows_v, trans_v,
              gsem0, gsem1, gsem2, gsem3, osem0, osem1, osem2, osem3):
    wid = lax.axis_index("s") * 2 + lax.axis_index("c")
    gsem = (gsem0, gsem1, gsem2, gsem3)
    osem = (osem0, osem1, osem2, osem3)

    pltpu.sync_copy(tok_hbm.at[pl.ds(0, _P), pl.ds(wid * _BPW, _BPW)], tok_v)

    def build_idx_and_fire(p, bt):
        pv = jnp.full((16,), p, dtype=jnp.int32)
        trow = tok_v.at[p]
        for s in range(8):
            t16 = trow[pl.ds(bt * 128 + s * 16, 16)]
            idx_v[bt, pl.ds(s * 16, 16)] = t16 * _P + pv
        pltpu.async_copy(fused_hbm.at[idx_v.at[bt]], rows_v.at[bt], gsem[bt])

    def wait_g(bt):
        pltpu.make_async_copy(
            fused_hbm.at[pl.ds(0, 128)], rows_v.at[bt], gsem[bt]).wait()

    lane = lax.iota(jnp.int32, 16)
    rot = [lax.bitwise_and(lane + k, 15) for k in range(16)]

    def transpose(bt):
        rows = rows_v.at[bt]
        tran = trans_v.at[bt]

        def bsub_body(bsub, carry):
            bvec = lane + bsub * 16
            for dsub in range(_D // 16):
                for k in range(16):
                    col = rot[k] + (dsub * 16)
                    v = plsc.load_gather(rows, [bvec, col])
                    tix = lax.shift_left(col, 7) + bvec
                    plsc.store_scatter(tran, [tix], v)
            return carry

        lax.fori_loop(0, 8, bsub_body, 0)

    def fire_out(p, bt):
        gbt = wid * 4 + bt
        for dblk in range(_D // 8):
            pltpu.async_copy(
                trans_v.at[bt].at[pl.ds(dblk * 1024, 1024)],
                out_hbm.at[p].at[dblk].at[gbt],
                osem[bt],
            )

    def wait_o(bt):
        for dblk in range(_D // 8):
            pltpu.make_async_copy(
                trans_v.at[bt].at[pl.ds(dblk * 1024, 1024)],
                out_hbm.at[0].at[dblk].at[0],
                osem[bt]).wait()

    for bt in range(4):
        build_idx_and_fire(0, bt)

    def body(p, carry):
        for bt in range(4):
            wait_g(bt)

            @pl.when(p >= 1)
            def _():
                wait_o(bt)

            transpose(bt)
            fire_out(p, bt)

            @pl.when(p <= _P - 2)
            def _():
                build_idx_and_fire(p + 1, bt)

        return carry

    lax.fori_loop(0, _P, body, 0)
    for bt in range(4):
        wait_o(bt)


def kernel(tokens, embedding, position_embedding):
    tok_t = tokens.astype(jnp.int32).T
    fused = _sc_fuse(embedding, position_embedding)
    out4 = _sc_embed(tok_t, fused)
    out5 = out4.reshape(_P, _D // 8, _B // 128, 8, 128)
    return out5.transpose(2, 4, 0, 1, 3).reshape(_B, _P, _D)


# speedup vs baseline: 5.0093x; 2.0436x over previous
import functools

import jax
import jax.numpy as jnp
from jax import lax
from jax.experimental import pallas as pl
from jax.experimental.pallas import tpu as pltpu
from jax.experimental.pallas import tpu_sc as plsc

_T = 1024
_P = 56
_D = 64
_B = 16384
_NW = 32
_BPW = _B // _NW
_NBLK = _P * (_BPW // 128)
_TPW = _T // _NW

_sc_mesh = plsc.VectorSubcoreMesh(core_axis_name="c", subcore_axis_name="s")
_sc_params = pltpu.CompilerParams(use_tc_tiling_on_sc=False,
                                  needs_layout_passes=False)


@functools.partial(
    pl.kernel,
    mesh=_sc_mesh,
    out_type=jax.ShapeDtypeStruct((_T * _P, _D), jnp.float32),
    scratch_types=[
        pltpu.VMEM((_TPW, _D), jnp.float32),
        pltpu.VMEM((_P, _D), jnp.float32),
        pltpu.VMEM((_TPW * _P, _D), jnp.float32),
    ],
    compiler_params=_sc_params,
)
def _sc_fuse(emb_hbm, pos_hbm, fused_hbm, emb_v, pos_v, f_v):
    wid = lax.axis_index("s") * 2 + lax.axis_index("c")
    pltpu.sync_copy(emb_hbm.at[pl.ds(wid * _TPW, _TPW)], emb_v)
    pltpu.sync_copy(pos_hbm, pos_v)

    def t_body(tl, carry):
        erow = emb_v.at[tl]
        e = [erow[pl.ds(i * 16, 16)] for i in range(_D // 16)]

        def p_body(p, c2):
            frow = f_v.at[tl * _P + p]
            prow = pos_v.at[p]
            for i in range(_D // 16):
                frow[pl.ds(i * 16, 16)] = e[i] + prow[pl.ds(i * 16, 16)]
            return c2

        lax.fori_loop(0, _P, p_body, 0)
        return carry

    lax.fori_loop(0, _TPW, t_body, 0)
    pltpu.sync_copy(f_v, fused_hbm.at[pl.ds(wid * _TPW * _P, _TPW * _P)])


@functools.partial(
    pl.kernel,
    mesh=_sc_mesh,
    out_type=jax.ShapeDtypeStruct((_P, _D // 8, _B // 128, 1024),
                                  jnp.float32),
    scratch_types=[
        pltpu.VMEM((_P, _BPW), jnp.int32),
        pltpu.VMEM((4, 128), jnp.int32),
        pltpu.VMEM((4, 128, _D), jnp.float32),
        pltpu.VMEM((4, _D * 128), jnp.float32),
        pltpu.SemaphoreType.DMA,
        pltpu.SemaphoreType.DMA,
        pltpu.SemaphoreType.DMA,
        pltpu.SemaphoreType.DMA,
        pltpu.SemaphoreType.DMA,
        pltpu.SemaphoreType.DMA,
        pltpu.SemaphoreType.DMA,
        pltpu.SemaphoreType.DMA,
    ],
    compiler_params=_sc_params,
)
def _sc_embed(tok_hbm, fused_hbm, out_hbm, tok_v, idx_v, rows_v, trans_v,
              gsem0, gsem1, gsem2, gsem3, osem0, osem1, osem2, osem3):
    wid = lax.axis_index("s") * 2 + lax.axis_index("c")
    gsem = (gsem0, gsem1, gsem2, gsem3)
    osem = (osem0, osem1, osem2, osem3)

    pltpu.sync_copy(tok_hbm.at[pl.ds(0, _P), pl.ds(wid * _BPW, _BPW)], tok_v)

    def build_idx_and_fire(p, bt):
        pv = jnp.full((16,), p, dtype=jnp.int32)
        trow = tok_v.at[p]
        for s in range(8):
            t16 = trow[pl.ds(bt * 128 + s * 16, 16)]
            idx_v[bt, pl.ds(s * 16, 16)] = t16 * _P + pv
        pltpu.async_copy(fused_hbm.at[idx_v.at[bt]], rows_v.at[bt], gsem[bt])

    def wait_g(bt):
        pltpu.make_async_copy(
            fused_hbm.at[pl.ds(0, 128)], rows_v.at[bt], gsem[bt]).wait()

    lane = lax.iota(jnp.int32, 16)
    rot = [lax.bitwise_and(lane + k, 15) for k in range(16)]

    def transpose(bt):
        rows = rows_v.at[bt]
        tran = trans_v.at[bt]

        def bsub_body(bsub, carry):
            bvec = lane + bsub * 16
            for dsub in range(_D // 16):
                vals, tixs = [], []
                for k in range(16):
                    col = rot[k] + (dsub * 16)
                    vals.append(plsc.load_gather(rows, [bvec, col]))
                    tixs.append(lax.shift_left(col, 7) + bvec)
                for k in range(16):
                    plsc.store_scatter(tran, [tixs[k]], vals[k])
            return carry

        lax.fori_loop(0, 8, bsub_body, 0)

    def fire_out(p, bt):
        gbt = wid * 4 + bt
        for dblk in range(_D // 8):
            pltpu.async_copy(
                trans_v.at[bt].at[pl.ds(dblk * 1024, 1024)],
                out_hbm.at[p].at[dblk].at[gbt],
                osem[bt],
            )

    def wait_o(bt):
        for dblk in range(_D // 8):
            pltpu.make_async_copy(
                trans_v.at[bt].at[pl.ds(dblk * 1024, 1024)],
                out_hbm.at[0].at[dblk].at[0],
                osem[bt]).wait()

    for bt in range(4):
        build_idx_and_fire(0, bt)

    def body(p, carry):
        for bt in range(4):
            wait_g(bt)

            @pl.when(p >= 1)
            def _():
                wait_o(bt)

            transpose(bt)
            fire_out(p, bt)

            @pl.when(p <= _P - 2)
            def _():
                build_idx_and_fire(p + 1, bt)

        return carry

    lax.fori_loop(0, _P, body, 0)
    for bt in range(4):
        wait_o(bt)


def kernel(tokens, embedding, position_embedding):
    tok_t = tokens.astype(jnp.int32).T
    fused = _sc_fuse(embedding, position_embedding)
    out4 = _sc_embed(tok_t, fused)
    out5 = out4.reshape(_P, _D // 8, _B // 128, 8, 128)
    return out5.transpose(2, 4, 0, 1, 3).reshape(_B, _P, _D)


# speedup vs baseline: 5.5231x; 1.1026x over previous
import functools

import jax
import jax.numpy as jnp
from jax import lax
from jax.experimental import pallas as pl
from jax.experimental.pallas import tpu as pltpu
from jax.experimental.pallas import tpu_sc as plsc

_T = 1024
_P = 56
_D = 64
_B = 16384
_NW = 32
_BPW = _B // _NW
_NBLK = _P * (_BPW // 128)
_TPW = _T // _NW

_sc_mesh = plsc.VectorSubcoreMesh(core_axis_name="c", subcore_axis_name="s")
_sc_params = pltpu.CompilerParams(use_tc_tiling_on_sc=False,
                                  needs_layout_passes=False)


@functools.partial(
    pl.kernel,
    mesh=_sc_mesh,
    out_type=jax.ShapeDtypeStruct((_T * _P, _D), jnp.float32),
    scratch_types=[
        pltpu.VMEM((_TPW, _D), jnp.float32),
        pltpu.VMEM((_P, _D), jnp.float32),
        pltpu.VMEM((_TPW * _P, _D), jnp.float32),
    ],
    compiler_params=_sc_params,
)
def _sc_fuse(emb_hbm, pos_hbm, fused_hbm, emb_v, pos_v, f_v):
    wid = lax.axis_index("s") * 2 + lax.axis_index("c")
    pltpu.sync_copy(emb_hbm.at[pl.ds(wid * _TPW, _TPW)], emb_v)
    pltpu.sync_copy(pos_hbm, pos_v)

    def p_body(p, carry):
        prow = pos_v.at[p]
        pv = [prow[pl.ds(i * 16, 16)] for i in range(_D // 16)]
        for tl in range(_TPW):
            erow = emb_v.at[tl]
            frow = f_v.at[tl * _P + p]
            for i in range(_D // 16):
                frow[pl.ds(i * 16, 16)] = erow[pl.ds(i * 16, 16)] + pv[i]
        return carry

    lax.fori_loop(0, _P, p_body, 0)
    pltpu.sync_copy(f_v, fused_hbm.at[pl.ds(wid * _TPW * _P, _TPW * _P)])


@functools.partial(
    pl.kernel,
    mesh=_sc_mesh,
    out_type=jax.ShapeDtypeStruct((_P, _D // 8, _B // 128, 1024),
                                  jnp.float32),
    scratch_types=[
        pltpu.VMEM((_P, _BPW), jnp.int32),
        pltpu.VMEM((4, 128), jnp.int32),
        pltpu.VMEM((4, 128, _D), jnp.float32),
        pltpu.VMEM((4, _D * 128), jnp.float32),
        pltpu.SemaphoreType.DMA,
        pltpu.SemaphoreType.DMA,
        pltpu.SemaphoreType.DMA,
        pltpu.SemaphoreType.DMA,
        pltpu.SemaphoreType.DMA,
        pltpu.SemaphoreType.DMA,
        pltpu.SemaphoreType.DMA,
        pltpu.SemaphoreType.DMA,
    ],
    compiler_params=_sc_params,
)
def _sc_embed(tok_hbm, fused_hbm, out_hbm, tok_v, idx_v, rows_v, trans_v,
              gsem0, gsem1, gsem2, gsem3, osem0, osem1, osem2, osem3):
    wid = lax.axis_index("s") * 2 + lax.axis_index("c")
    gsem = (gsem0, gsem1, gsem2, gsem3)
    osem = (osem0, osem1, osem2, osem3)

    pltpu.sync_copy(tok_hbm.at[pl.ds(0, _P), pl.ds(wid * _BPW, _BPW)], tok_v)

    def build_idx_and_fire(p, bt):
        pv = jnp.full((16,), p, dtype=jnp.int32)
        trow = tok_v.at[p]
        for s in range(8):
            t16 = trow[pl.ds(bt * 128 + s * 16, 16)]
            idx_v[bt, pl.ds(s * 16, 16)] = t16 * _P + pv
        pltpu.async_copy(fused_hbm.at[idx_v.at[bt]], rows_v.at[bt], gsem[bt])

    def wait_g(bt):
        pltpu.make_async_copy(
            fused_hbm.at[pl.ds(0, 128)], rows_v.at[bt], gsem[bt]).wait()

    lane = lax.iota(jnp.int32, 16)
    rot = [lax.bitwise_and(lane + k, 15) for k in range(16)]

    def transpose(bt):
        rows = rows_v.at[bt]
        tran = trans_v.at[bt]

        def bsub_body(bsub, carry):
            bvec = lane + bsub * 16
            for dsub in range(_D // 16):
                for g in range(2):
                    vals, cols = [], []
                    for k in range(8):
                        col = rot[g * 8 + k] + (dsub * 16)
                        cols.append(col)
                        vals.append(plsc.load_gather(rows, [bvec, col]))
                    for k in range(8):
                        tix = lax.shift_left(cols[k], 7) + bvec
                        plsc.store_scatter(tran, [tix], vals[k])
            return carry

        lax.fori_loop(0, 8, bsub_body, 0)

    def fire_out(p, bt):
        gbt = wid * 4 + bt
        for dblk in range(_D // 8):
            pltpu.async_copy(
                trans_v.at[bt].at[pl.ds(dblk * 1024, 1024)],
                out_hbm.at[p].at[dblk].at[gbt],
                osem[bt],
            )

    def wait_o(bt):
        for dblk in range(_D // 8):
            pltpu.make_async_copy(
                trans_v.at[bt].at[pl.ds(dblk * 1024, 1024)],
                out_hbm.at[0].at[dblk].at[0],
                osem[bt]).wait()

    for bt in range(4):
        build_idx_and_fire(0, bt)

    def body(p, carry):
        for bt in range(4):
            wait_g(bt)

            @pl.when(p >= 1)
            def _():
                wait_o(bt)

            transpose(bt)
            fire_out(p, bt)

            @pl.when(p <= _P - 2)
            def _():
                build_idx_and_fire(p + 1, bt)

        return carry

    lax.fori_loop(0, _P, body, 0)
    for bt in range(4):
        wait_o(bt)


def kernel(tokens, embedding, position_embedding):
    tok_t = tokens.astype(jnp.int32).T
    fused = _sc_fuse(embedding, position_embedding)
    out4 = _sc_embed(tok_t, fused)
    out5 = out4.reshape(_P, _D // 8, _B // 128, 8, 128)
    return out5.transpose(2, 4, 0, 1, 3).reshape(_B, _P, _D)


# speedup vs baseline: 6.0505x; 1.0955x over previous
import functools

import jax
import jax.numpy as jnp
from jax import lax
from jax.experimental import pallas as pl
from jax.experimental.pallas import tpu as pltpu
from jax.experimental.pallas import tpu_sc as plsc

_T = 1024
_P = 56
_D = 64
_B = 16384
_NW = 32
_BPW = _B // _NW
_NBLK = _P * (_BPW // 128)
_TPW = _T // _NW

_sc_mesh = plsc.VectorSubcoreMesh(core_axis_name="c", subcore_axis_name="s")
_sc_params = pltpu.CompilerParams(use_tc_tiling_on_sc=False,
                                  needs_layout_passes=False)


@functools.partial(
    pl.kernel,
    mesh=_sc_mesh,
    out_type=jax.ShapeDtypeStruct((_T * _P, _D), jnp.float32),
    scratch_types=[
        pltpu.VMEM((_TPW, _D), jnp.float32),
        pltpu.VMEM((_P, _D), jnp.float32),
        pltpu.VMEM((_TPW * _P, _D), jnp.float32),
    ],
    compiler_params=_sc_params,
)
def _sc_fuse(emb_hbm, pos_hbm, fused_hbm, emb_v, pos_v, f_v):
    wid = lax.axis_index("s") * 2 + lax.axis_index("c")
    pltpu.sync_copy(emb_hbm.at[pl.ds(wid * _TPW, _TPW)], emb_v)
    pltpu.sync_copy(pos_hbm, pos_v)

    def p_body(p, carry):
        prow = pos_v.at[p]
        pv = [prow[pl.ds(i * 16, 16)] for i in range(_D // 16)]
        for tl in range(_TPW):
            erow = emb_v.at[tl]
            frow = f_v.at[tl * _P + p]
            for i in range(_D // 16):
                frow[pl.ds(i * 16, 16)] = erow[pl.ds(i * 16, 16)] + pv[i]
        return carry

    lax.fori_loop(0, _P, p_body, 0)
    pltpu.sync_copy(f_v, fused_hbm.at[pl.ds(wid * _TPW * _P, _TPW * _P)])


@functools.partial(
    pl.kernel,
    mesh=_sc_mesh,
    out_type=jax.ShapeDtypeStruct((_P, _D // 8, _NW, 4096), jnp.float32),
    scratch_types=[
        pltpu.VMEM((_P, _BPW), jnp.int32),
        pltpu.VMEM((4, 64), jnp.int32),
        pltpu.VMEM((4, 64, _D), jnp.float32),
        pltpu.VMEM((2, (_D // 8) * 4096), jnp.float32),
        pltpu.SemaphoreType.DMA,
        pltpu.SemaphoreType.DMA,
        pltpu.SemaphoreType.DMA,
        pltpu.SemaphoreType.DMA,
        pltpu.SemaphoreType.DMA,
        pltpu.SemaphoreType.DMA,
    ],
    compiler_params=_sc_params,
)
def _sc_embed(tok_hbm, fused_hbm, out_hbm, tok_v, idx_v, rows_v, trans_v,
              gsem0, gsem1, gsem2, gsem3, osem0, osem1):
    wid = lax.axis_index("s") * 2 + lax.axis_index("c")
    gsem = (gsem0, gsem1, gsem2, gsem3)
    osem = (osem0, osem1)

    pltpu.sync_copy(tok_hbm.at[pl.ds(0, _P), pl.ds(wid * _BPW, _BPW)], tok_v)

    def build_idx_and_fire(p, sb):
        slot = sb % 4
        pv = jnp.full((16,), p, dtype=jnp.int32)
        trow = tok_v.at[p]
        for s in range(4):
            t16 = trow[pl.ds(sb * 64 + s * 16, 16)]
            idx_v[slot, pl.ds(s * 16, 16)] = t16 * _P + pv
        pltpu.async_copy(fused_hbm.at[idx_v.at[slot]], rows_v.at[slot],
                         gsem[slot])

    def wait_g(slot):
        pltpu.make_async_copy(
            fused_hbm.at[pl.ds(0, 64)], rows_v.at[slot], gsem[slot]).wait()

    lane = lax.iota(jnp.int32, 16)
    rot = [lax.bitwise_and(lane + k, 15) for k in range(16)]
    rst = [
        lax.shift_left(lax.shift_right_logical(rot[k], 3), 12)
        + lax.shift_left(lax.bitwise_and(rot[k], 7), 7)
        for k in range(16)
    ]

    def transpose(pb, sb):
        rows = rows_v.at[sb % 4]
        tran = trans_v.at[pb]
        boff = (sb // 2) * 1024 + (sb & 1) * 64

        def bsub_body(i, carry):
            bsub = lax.bitwise_and(i, 3)
            dsub = lax.shift_right_logical(i, 2)
            bvec = lane + bsub * 16
            d0 = dsub * 16
            sbase = bvec + (lax.shift_right_logical(d0, 3) * 4096 + boff)
            for g in range(2):
                vals = []
                for k in range(8):
                    col = rot[g * 8 + k] + d0
                    vals.append(plsc.load_gather(rows, [bvec, col]))
                for k in range(8):
                    tix = rst[g * 8 + k] + sbase
                    plsc.store_scatter(tran, [tix], vals[k])
            return carry

        lax.fori_loop(0, 16, bsub_body, 0)

    def fire_out(p, pb):
        for dblk in range(_D // 8):
            pltpu.async_copy(
                trans_v.at[pb].at[pl.ds(dblk * 4096, 4096)],
                out_hbm.at[p].at[dblk].at[wid],
                osem[pb],
            )

    def wait_o(pb):
        for dblk in range(_D // 8):
            pltpu.make_async_copy(
                trans_v.at[pb].at[pl.ds(dblk * 4096, 4096)],
                out_hbm.at[0].at[dblk].at[0],
                osem[pb]).wait()

    for sb in range(4):
        build_idx_and_fire(0, sb)

    def body(jj, carry):
        for pb in (0, 1):
            p = 2 * jj + pb

            @pl.when(jj >= 1)
            def _():
                wait_o(pb)

            for sb in range(8):
                wait_g(sb % 4)
                transpose(pb, sb)
                if sb < 4:
                    build_idx_and_fire(p, sb + 4)
                elif pb == 0:
                    build_idx_and_fire(p + 1, sb - 4)
                else:
                    @pl.when(jj <= _P // 2 - 2)
                    def _():
                        build_idx_and_fire(p + 1, sb - 4)

            fire_out(p, pb)

        return carry

    lax.fori_loop(0, _P // 2, body, 0)
    wait_o(0)
    wait_o(1)


def kernel(tokens, embedding, position_embedding):
    tok_t = tokens.astype(jnp.int32).T
    fused = _sc_fuse(embedding, position_embedding)
    outw = _sc_embed(tok_t, fused)
    out5 = outw.reshape(_P, _D // 8, _B // 128, 8, 128)
    return out5.transpose(2, 4, 0, 1, 3).reshape(_B, _P, _D)
